# Initial kernel scaffold; baseline (speedup 1.0000x reference)
import jax
import jax.numpy as jnp
from jax.experimental import pallas as pl


def kernel(poses_rest, edge_meas, edge_weights, edges_src, edges_dst):
    raise NotImplementedError("write your pallas kernel here")


# speedup vs baseline: 3.2147x; 3.2147x over previous
import functools

import jax
import jax.numpy as jnp
from jax import lax
from jax.experimental import pallas as pl
from jax.experimental.pallas import tpu as pltpu
from jax.experimental.pallas import tpu_sc as plsc

N_POSES = 100000
N_EDGES = 1600000
CHUNK = 512
N_CHUNKS = N_EDGES // CHUNK
N_WORKERS = 32
GATHER_SUB = 128
LANES = 16

_ATAN_C = (
    0.9999999227745342,
    -0.33332232446552534,
    0.19974024787337796,
    -0.1404779314760279,
    0.10002110151363129,
    -0.060872867152485396,
    0.025330362663490005,
    -0.005020633421971556,
)


def _rsqrt(x):
    i = plsc.bitcast(x, jnp.int32)
    i = jnp.int32(0x5F3759DF) - lax.shift_right_logical(i, 1)
    y = plsc.bitcast(i, jnp.float32)
    half = 0.5 * x
    for _ in range(3):
        y = y * (1.5 - half * y * y)
    return y


def _atan01(a):
    u = a * a
    p = jnp.full_like(a, _ATAN_C[-1])
    for c in _ATAN_C[-2::-1]:
        p = p * u + c
    return a * p


def _cross(ax, ay, az, bx, by, bz):
    return (ay * bz - az * by, az * bx - ax * bz, ax * by - ay * bx)


def _quat_rotate(qx, qy, qz, qw, vx, vy, vz):
    tx, ty, tz = _cross(qx, qy, qz, vx, vy, vz)
    tx, ty, tz = 2.0 * tx, 2.0 * ty, 2.0 * tz
    cx, cy, cz = _cross(qx, qy, qz, tx, ty, tz)
    return (vx + qw * tx + cx, vy + qw * ty + cy, vz + qw * tz + cz)


def _quat_mul(x1, y1, z1, w1, x2, y2, z2, w2):
    x = w1 * x2 + x1 * w2 + y1 * z2 - z1 * y2
    y = w1 * y2 - x1 * z2 + y1 * w2 + z1 * x2
    z = w1 * z2 + x1 * y2 - y1 * x2 + z1 * w2
    w = w1 * w2 - x1 * x2 - y1 * y2 - z1 * z2
    return x, y, z, w


def _sc_body(table, src_idx, dst_idx, meas, wts, out,
             si_v, di_v, src_v, dst_v, meas_v, w_v, out_v, sem):
    wid = lax.axis_index("s") * 2 + lax.axis_index("c")
    n_chunks = (N_CHUNKS - wid + N_WORKERS - 1) // N_WORKERS
    lane = lax.iota(jnp.int32, LANES)

    def chunk_body(i, carry):
        c = wid + i * N_WORKERS
        base = c * CHUNK

        pltpu.sync_copy(src_idx.at[pl.ds(base, CHUNK)], si_v)
        pltpu.sync_copy(dst_idx.at[pl.ds(base, CHUNK)], di_v)
        descs = []
        for j in range(CHUNK // GATHER_SUB):
            sl = pl.ds(j * GATHER_SUB, GATHER_SUB)
            descs.append(pltpu.async_copy(table.at[si_v.at[sl]], src_v.at[sl], sem))
            descs.append(pltpu.async_copy(table.at[di_v.at[sl]], dst_v.at[sl], sem))
        pltpu.sync_copy(meas.at[pl.ds(base, CHUNK), :], meas_v)
        pltpu.sync_copy(wts.at[pl.ds(base, CHUNK), :], w_v)
        for d in descs:
            d.wait()

        def group_body(g, carry2):
            rows = g * LANES + lane

            def col(ref, k):
                return plsc.load_gather(ref, [rows, jnp.full((LANES,), k, jnp.int32)])

            stx, sty, stz = col(src_v, 0), col(src_v, 1), col(src_v, 2)
            sqx, sqy, sqz, sqw = col(src_v, 3), col(src_v, 4), col(src_v, 5), col(src_v, 6)
            dtx, dty, dtz = col(dst_v, 0), col(dst_v, 1), col(dst_v, 2)
            dqx, dqy, dqz, dqw = col(dst_v, 3), col(dst_v, 4), col(dst_v, 5), col(dst_v, 6)
            mtx, mty, mtz = col(meas_v, 0), col(meas_v, 1), col(meas_v, 2)
            mqx, mqy, mqz, mqw = col(meas_v, 3), col(meas_v, 4), col(meas_v, 5), col(meas_v, 6)

            iqx, iqy, iqz = -sqx, -sqy, -sqz
            rx, ry, rz = _quat_rotate(iqx, iqy, iqz, sqw, stx, sty, stz)
            itx, ity, itz = -rx, -ry, -rz
            rdx, rdy, rdz = _quat_rotate(iqx, iqy, iqz, sqw, dtx, dty, dtz)
            ptx, pty, ptz = itx + rdx, ity + rdy, itz + rdz
            pqx, pqy, pqz, pqw = _quat_mul(iqx, iqy, iqz, sqw, dqx, dqy, dqz, dqw)
            jqx, jqy, jqz = -mqx, -mqy, -mqz
            mx, my, mz = _quat_rotate(jqx, jqy, jqz, mqw, mtx, mty, mtz)
            ntx, nty, ntz = -mx, -my, -mz
            rpx, rpy, rpz = _quat_rotate(jqx, jqy, jqz, mqw, ptx, pty, ptz)
            tx, ty, tz = ntx + rpx, nty + rpy, ntz + rpz
            qx, qy, qz, qw = _quat_mul(jqx, jqy, jqz, mqw, pqx, pqy, pqz, pqw)

            sign = jnp.where(qw < 0.0, -1.0, 1.0)
            qx, qy, qz, qw = qx * sign, qy * sign, qz * sign, qw * sign
            n2 = qx * qx + qy * qy + qz * qz
            n = (n2 + 1e-24) * _rsqrt(n2 + 1e-24)
            mn = jnp.minimum(n, qw)
            mx_ = jnp.maximum(n, qw)
            t_at = _atan01(mn / mx_)
            half = jnp.where(n > qw, jnp.float32(jnp.pi / 2) - t_at, t_at)
            theta = 2.0 * half
            small = n < 1e-6
            fnum = jnp.where(small, 2.0, theta)
            fden = jnp.where(small, jnp.maximum(qw, 1e-6), n)
            factor = fnum / fden
            phx, phy, phz = factor * qx, factor * qy, factor * qz

            th2 = phx * phx + phy * phy + phz * phz
            th2_safe = jnp.where(th2 < 1e-12, 1.0, th2)
            cot_term = half * qw / jnp.maximum(n, 1e-8)
            coef = jnp.where(theta < 1e-4, jnp.float32(1.0 / 12.0),
                             (1.0 - cot_term) / th2_safe)
            p1x, p1y, p1z = _cross(phx, phy, phz, tx, ty, tz)
            p2x, p2y, p2z = _cross(phx, phy, phz, p1x, p1y, p1z)
            taux = tx - 0.5 * p1x + coef * p2x
            tauy = ty - 0.5 * p1y + coef * p2y
            tauz = tz - 0.5 * p1z + coef * p2z

            obase = g * (LANES * 6) + lane * 6
            vals = (taux * col(w_v, 0), tauy * col(w_v, 1), tauz * col(w_v, 2),
                    phx * col(w_v, 3), phy * col(w_v, 4), phz * col(w_v, 5))
            for k, v in enumerate(vals):
                plsc.store_scatter(out_v, [obase + k], v)
            return carry2

        lax.fori_loop(0, CHUNK // LANES, group_body, 0, unroll=False)
        pltpu.sync_copy(out_v, out.at[pl.ds(base * 6, CHUNK * 6)])
        return carry

    lax.fori_loop(0, n_chunks, chunk_body, 0, unroll=False)


@jax.jit
def _pose_graph_sc(table, src_idx, dst_idx, meas, wts):
    mesh = plsc.VectorSubcoreMesh(core_axis_name="c", subcore_axis_name="s")
    f = pl.kernel(
        _sc_body,
        out_type=jax.ShapeDtypeStruct((N_EDGES * 6,), jnp.float32),
        mesh=mesh,
        compiler_params=pltpu.CompilerParams(
            needs_layout_passes=False, use_tc_tiling_on_sc=False),
        scratch_types=[
            pltpu.VMEM((CHUNK,), jnp.int32),
            pltpu.VMEM((CHUNK,), jnp.int32),
            pltpu.VMEM((CHUNK, 8), jnp.float32),
            pltpu.VMEM((CHUNK, 8), jnp.float32),
            pltpu.VMEM((CHUNK, 7), jnp.float32),
            pltpu.VMEM((CHUNK, 6), jnp.float32),
            pltpu.VMEM((CHUNK * 6,), jnp.float32),
            pltpu.SemaphoreType.DMA,
        ],
    )
    return f(table, src_idx, dst_idx, meas, wts)


def kernel(poses_rest, edge_meas, edge_weights, edges_src, edges_dst):
    pose_0 = jnp.zeros((1, 7), jnp.float32).at[0, 6].set(1.0)
    all_poses = jnp.concatenate([pose_0, poses_rest], axis=0)
    table = jnp.pad(all_poses, ((0, 0), (0, 1)))
    return _pose_graph_sc(
        table,
        edges_src.astype(jnp.int32),
        edges_dst.astype(jnp.int32),
        edge_meas,
        edge_weights,
    )


# speedup vs baseline: 3.3522x; 1.0428x over previous
import functools

import jax
import jax.numpy as jnp
from jax import lax
from jax.experimental import pallas as pl
from jax.experimental.pallas import tpu as pltpu
from jax.experimental.pallas import tpu_sc as plsc

N_POSES = 100000
N_EDGES = 1600000
CHUNK = 512
N_CHUNKS = N_EDGES // CHUNK
N_WORKERS = 32
GATHER_SUB = 128
LANES = 16

_ATAN_C = (
    0.9999999227745342,
    -0.33332232446552534,
    0.19974024787337796,
    -0.1404779314760279,
    0.10002110151363129,
    -0.060872867152485396,
    0.025330362663490005,
    -0.005020633421971556,
)


def _rsqrt(x):
    i = plsc.bitcast(x, jnp.int32)
    i = jnp.int32(0x5F3759DF) - lax.shift_right_logical(i, 1)
    y = plsc.bitcast(i, jnp.float32)
    half = 0.5 * x
    for _ in range(3):
        y = y * (1.5 - half * y * y)
    return y


def _atan01(a):
    u = a * a
    p = jnp.full_like(a, _ATAN_C[-1])
    for c in _ATAN_C[-2::-1]:
        p = p * u + c
    return a * p


def _cross(ax, ay, az, bx, by, bz):
    return (ay * bz - az * by, az * bx - ax * bz, ax * by - ay * bx)


def _quat_rotate(qx, qy, qz, qw, vx, vy, vz):
    tx, ty, tz = _cross(qx, qy, qz, vx, vy, vz)
    tx, ty, tz = 2.0 * tx, 2.0 * ty, 2.0 * tz
    cx, cy, cz = _cross(qx, qy, qz, tx, ty, tz)
    return (vx + qw * tx + cx, vy + qw * ty + cy, vz + qw * tz + cz)


def _quat_mul(x1, y1, z1, w1, x2, y2, z2, w2):
    x = w1 * x2 + x1 * w2 + y1 * z2 - z1 * y2
    y = w1 * y2 - x1 * z2 + y1 * w2 + z1 * x2
    z = w1 * z2 + x1 * y2 - y1 * x2 + z1 * w2
    w = w1 * w2 - x1 * x2 - y1 * y2 - z1 * z2
    return x, y, z, w


def _sc_body(table, src_idx, dst_idx, meas, wts, out,
             si_v, di_v, src_v, dst_v, meas_v, w_v, out_v, sem):
    wid = lax.axis_index("s") * 2 + lax.axis_index("c")
    n_chunks = (N_CHUNKS - wid + N_WORKERS - 1) // N_WORKERS
    lane = lax.iota(jnp.int32, LANES)

    def chunk_body(i, carry):
        c = wid + i * N_WORKERS
        base = c * CHUNK

        pltpu.sync_copy(src_idx.at[pl.ds(base, CHUNK)], si_v)
        pltpu.sync_copy(dst_idx.at[pl.ds(base, CHUNK)], di_v)
        descs = []
        for j in range(CHUNK // GATHER_SUB):
            sl = pl.ds(j * GATHER_SUB, GATHER_SUB)
            descs.append(pltpu.async_copy(table.at[si_v.at[sl]], src_v.at[sl], sem))
            descs.append(pltpu.async_copy(table.at[di_v.at[sl]], dst_v.at[sl], sem))
        for k in range(7):
            pltpu.sync_copy(meas.at[pl.ds(k * N_EDGES + base, CHUNK)],
                            meas_v.at[k])
        for k in range(6):
            pltpu.sync_copy(wts.at[pl.ds(k * N_EDGES + base, CHUNK)],
                            w_v.at[k])
        for d in descs:
            d.wait()

        def group_body(g, carry2):
            rows = g * LANES + lane

            def col(ref, k):
                return plsc.load_gather(ref, [rows, jnp.full((LANES,), k, jnp.int32)])

            def cols(ref, k):
                return ref[k, pl.ds(g * LANES, LANES)]

            stx, sty, stz = col(src_v, 0), col(src_v, 1), col(src_v, 2)
            sqx, sqy, sqz, sqw = col(src_v, 3), col(src_v, 4), col(src_v, 5), col(src_v, 6)
            dtx, dty, dtz = col(dst_v, 0), col(dst_v, 1), col(dst_v, 2)
            dqx, dqy, dqz, dqw = col(dst_v, 3), col(dst_v, 4), col(dst_v, 5), col(dst_v, 6)
            mtx, mty, mtz = cols(meas_v, 0), cols(meas_v, 1), cols(meas_v, 2)
            mqx, mqy, mqz, mqw = (cols(meas_v, 3), cols(meas_v, 4),
                                  cols(meas_v, 5), cols(meas_v, 6))

            iqx, iqy, iqz = -sqx, -sqy, -sqz
            rx, ry, rz = _quat_rotate(iqx, iqy, iqz, sqw, stx, sty, stz)
            itx, ity, itz = -rx, -ry, -rz
            rdx, rdy, rdz = _quat_rotate(iqx, iqy, iqz, sqw, dtx, dty, dtz)
            ptx, pty, ptz = itx + rdx, ity + rdy, itz + rdz
            pqx, pqy, pqz, pqw = _quat_mul(iqx, iqy, iqz, sqw, dqx, dqy, dqz, dqw)
            jqx, jqy, jqz = -mqx, -mqy, -mqz
            mx, my, mz = _quat_rotate(jqx, jqy, jqz, mqw, mtx, mty, mtz)
            ntx, nty, ntz = -mx, -my, -mz
            rpx, rpy, rpz = _quat_rotate(jqx, jqy, jqz, mqw, ptx, pty, ptz)
            tx, ty, tz = ntx + rpx, nty + rpy, ntz + rpz
            qx, qy, qz, qw = _quat_mul(jqx, jqy, jqz, mqw, pqx, pqy, pqz, pqw)

            sign = jnp.where(qw < 0.0, -1.0, 1.0)
            qx, qy, qz, qw = qx * sign, qy * sign, qz * sign, qw * sign
            n2 = qx * qx + qy * qy + qz * qz
            n = (n2 + 1e-24) * _rsqrt(n2 + 1e-24)
            mn = jnp.minimum(n, qw)
            mx_ = jnp.maximum(n, qw)
            t_at = _atan01(mn / mx_)
            half = jnp.where(n > qw, jnp.float32(jnp.pi / 2) - t_at, t_at)
            theta = 2.0 * half
            small = n < 1e-6
            fnum = jnp.where(small, 2.0, theta)
            fden = jnp.where(small, jnp.maximum(qw, 1e-6), n)
            factor = fnum / fden
            phx, phy, phz = factor * qx, factor * qy, factor * qz

            th2 = phx * phx + phy * phy + phz * phz
            th2_safe = jnp.where(th2 < 1e-12, 1.0, th2)
            cot_term = half * qw / jnp.maximum(n, 1e-8)
            coef = jnp.where(theta < 1e-4, jnp.float32(1.0 / 12.0),
                             (1.0 - cot_term) / th2_safe)
            p1x, p1y, p1z = _cross(phx, phy, phz, tx, ty, tz)
            p2x, p2y, p2z = _cross(phx, phy, phz, p1x, p1y, p1z)
            taux = tx - 0.5 * p1x + coef * p2x
            tauy = ty - 0.5 * p1y + coef * p2y
            tauz = tz - 0.5 * p1z + coef * p2z

            obase = g * (LANES * 6) + lane * 6
            vals = (taux * cols(w_v, 0), tauy * cols(w_v, 1), tauz * cols(w_v, 2),
                    phx * cols(w_v, 3), phy * cols(w_v, 4), phz * cols(w_v, 5))
            for k, v in enumerate(vals):
                plsc.store_scatter(out_v, [obase + k], v)
            return carry2

        lax.fori_loop(0, CHUNK // LANES, group_body, 0, unroll=False)
        pltpu.sync_copy(out_v, out.at[pl.ds(base * 6, CHUNK * 6)])
        return carry

    lax.fori_loop(0, n_chunks, chunk_body, 0, unroll=False)


@jax.jit
def _pose_graph_sc(table, src_idx, dst_idx, meas, wts):
    mesh = plsc.VectorSubcoreMesh(core_axis_name="c", subcore_axis_name="s")
    f = pl.kernel(
        _sc_body,
        out_type=jax.ShapeDtypeStruct((N_EDGES * 6,), jnp.float32),
        mesh=mesh,
        compiler_params=pltpu.CompilerParams(
            needs_layout_passes=False, use_tc_tiling_on_sc=False),
        scratch_types=[
            pltpu.VMEM((CHUNK,), jnp.int32),
            pltpu.VMEM((CHUNK,), jnp.int32),
            pltpu.VMEM((CHUNK, 8), jnp.float32),
            pltpu.VMEM((CHUNK, 8), jnp.float32),
            pltpu.VMEM((7, CHUNK), jnp.float32),
            pltpu.VMEM((6, CHUNK), jnp.float32),
            pltpu.VMEM((CHUNK * 6,), jnp.float32),
            pltpu.SemaphoreType.DMA,
        ],
    )
    return f(table, src_idx, dst_idx, meas, wts)


def kernel(poses_rest, edge_meas, edge_weights, edges_src, edges_dst):
    pose_0 = jnp.zeros((1, 7), jnp.float32).at[0, 6].set(1.0)
    all_poses = jnp.concatenate([pose_0, poses_rest], axis=0)
    table = jnp.pad(all_poses, ((0, 0), (0, 1)))
    return _pose_graph_sc(
        table,
        edges_src.astype(jnp.int32),
        edges_dst.astype(jnp.int32),
        edge_meas.T.reshape(-1),
        edge_weights.T.reshape(-1),
    )


# speedup vs baseline: 18.3406x; 5.4712x over previous
import jax
import jax.numpy as jnp
from jax import lax
from jax.experimental import pallas as pl
from jax.experimental.pallas import tpu as pltpu
from jax.experimental.pallas import tpu_sc as plsc

N_POSES = 100000
N_EDGES = 1600000
CHUNK = 512
N_CHUNKS = N_EDGES // CHUNK
N_WORKERS = 32
T_ITER = (N_CHUNKS + N_WORKERS - 1) // N_WORKERS
GATHER_SUB = 128
LANES = 16
GROUPS = CHUNK // LANES

_ATAN_C = (
    0.9999999227745342,
    -0.33332232446552534,
    0.19974024787337796,
    -0.1404779314760279,
    0.10002110151363129,
    -0.060872867152485396,
    0.025330362663490005,
    -0.005020633421971556,
)


def _rsqrt(x):
    i = plsc.bitcast(x, jnp.int32)
    i = jnp.int32(0x5F3759DF) - lax.shift_right_logical(i, 1)
    y = plsc.bitcast(i, jnp.float32)
    half = 0.5 * x
    for _ in range(3):
        y = y * (1.5 - half * y * y)
    return y


def _atan01(a):
    u = a * a
    p = jnp.full_like(a, _ATAN_C[-1])
    for c in _ATAN_C[-2::-1]:
        p = p * u + c
    return a * p


def _cross(ax, ay, az, bx, by, bz):
    return (ay * bz - az * by, az * bx - ax * bz, ax * by - ay * bx)


def _quat_rotate(qx, qy, qz, qw, vx, vy, vz):
    tx, ty, tz = _cross(qx, qy, qz, vx, vy, vz)
    tx, ty, tz = 2.0 * tx, 2.0 * ty, 2.0 * tz
    cx, cy, cz = _cross(qx, qy, qz, tx, ty, tz)
    return (vx + qw * tx + cx, vy + qw * ty + cy, vz + qw * tz + cz)


def _quat_mul(x1, y1, z1, w1, x2, y2, z2, w2):
    x = w1 * x2 + x1 * w2 + y1 * z2 - z1 * y2
    y = w1 * y2 - x1 * z2 + y1 * w2 + z1 * x2
    z = w1 * z2 + x1 * y2 - y1 * x2 + z1 * w2
    w = w1 * w2 - x1 * x2 - y1 * y2 - z1 * z2
    return x, y, z, w


def _sc_body(table, src_idx, dst_idx, meas, wts, out,
             si0, si1, di0, di1, sv0, sv1, dv0, dv1,
             mv0, mv1, wv0, wv1, ov0, ov1,
             ix0, ix1, gt0, gt1, mw0, mw1, st0, st1):
    SI, DI = (si0, si1), (di0, di1)
    SV, DV = (sv0, sv1), (dv0, dv1)
    MV, WV = (mv0, mv1), (wv0, wv1)
    OV = (ov0, ov1)
    IX, GT, MW, ST = (ix0, ix1), (gt0, gt1), (mw0, mw1), (st0, st1)

    wid = lax.axis_index("s") * 2 + lax.axis_index("c")
    lane = lax.iota(jnp.int32, LANES)

    def cbase(i):
        c = jnp.minimum(wid + i * N_WORKERS, N_CHUNKS - 1)
        return c * CHUNK

    def issue_idx(i, p):
        base = cbase(i)
        pltpu.async_copy(src_idx.at[pl.ds(base, CHUNK)], SI[p], IX[p])
        pltpu.async_copy(dst_idx.at[pl.ds(base, CHUNK)], DI[p], IX[p])

    def wait_idx(p):
        pltpu.make_async_copy(src_idx.at[pl.ds(0, CHUNK)], SI[p], IX[p]).wait()
        pltpu.make_async_copy(dst_idx.at[pl.ds(0, CHUNK)], DI[p], IX[p]).wait()

    def issue_gathers(i, p):
        for j in range(CHUNK // GATHER_SUB):
            sl = pl.ds(j * GATHER_SUB, GATHER_SUB)
            pltpu.async_copy(table.at[SI[p].at[sl]], SV[p].at[sl], GT[p])
            pltpu.async_copy(table.at[DI[p].at[sl]], DV[p].at[sl], GT[p])

    def wait_gathers(p):
        for j in range(CHUNK // GATHER_SUB):
            sl = pl.ds(j * GATHER_SUB, GATHER_SUB)
            pltpu.make_async_copy(table.at[pl.ds(0, GATHER_SUB)], SV[p].at[sl], GT[p]).wait()
            pltpu.make_async_copy(table.at[pl.ds(0, GATHER_SUB)], DV[p].at[sl], GT[p]).wait()

    def issue_mw(i, p):
        base = cbase(i)
        for k in range(7):
            pltpu.async_copy(meas.at[pl.ds(k * N_EDGES + base, CHUNK)],
                             MV[p].at[k], MW[p])
        for k in range(6):
            pltpu.async_copy(wts.at[pl.ds(k * N_EDGES + base, CHUNK)],
                             WV[p].at[k], MW[p])

    def wait_mw(p):
        for k in range(7):
            pltpu.make_async_copy(meas.at[pl.ds(0, CHUNK)], MV[p].at[k], MW[p]).wait()
        for k in range(6):
            pltpu.make_async_copy(wts.at[pl.ds(0, CHUNK)], WV[p].at[k], MW[p]).wait()

    def issue_store(i, p):
        pltpu.async_copy(OV[p], out.at[pl.ds(cbase(i) * 6, CHUNK * 6)], ST[p])

    def wait_store(p):
        pltpu.make_async_copy(out.at[pl.ds(0, CHUNK * 6)], OV[p], ST[p]).wait()

    def compute(p):
        src_v, dst_v, meas_v, w_v, out_v = SV[p], DV[p], MV[p], WV[p], OV[p]

        def group_body(g, carry2):
            rows = g * LANES + lane

            def col(ref, k):
                return plsc.load_gather(ref, [rows, jnp.full((LANES,), k, jnp.int32)])

            def cols(ref, k):
                return ref[k, pl.ds(g * LANES, LANES)]

            stx, sty, stz = col(src_v, 0), col(src_v, 1), col(src_v, 2)
            sqx, sqy, sqz, sqw = col(src_v, 3), col(src_v, 4), col(src_v, 5), col(src_v, 6)
            dtx, dty, dtz = col(dst_v, 0), col(dst_v, 1), col(dst_v, 2)
            dqx, dqy, dqz, dqw = col(dst_v, 3), col(dst_v, 4), col(dst_v, 5), col(dst_v, 6)
            mtx, mty, mtz = cols(meas_v, 0), cols(meas_v, 1), cols(meas_v, 2)
            mqx, mqy, mqz, mqw = (cols(meas_v, 3), cols(meas_v, 4),
                                  cols(meas_v, 5), cols(meas_v, 6))

            iqx, iqy, iqz = -sqx, -sqy, -sqz
            rx, ry, rz = _quat_rotate(iqx, iqy, iqz, sqw, stx, sty, stz)
            itx, ity, itz = -rx, -ry, -rz
            rdx, rdy, rdz = _quat_rotate(iqx, iqy, iqz, sqw, dtx, dty, dtz)
            ptx, pty, ptz = itx + rdx, ity + rdy, itz + rdz
            pqx, pqy, pqz, pqw = _quat_mul(iqx, iqy, iqz, sqw, dqx, dqy, dqz, dqw)
            jqx, jqy, jqz = -mqx, -mqy, -mqz
            mx, my, mz = _quat_rotate(jqx, jqy, jqz, mqw, mtx, mty, mtz)
            ntx, nty, ntz = -mx, -my, -mz
            rpx, rpy, rpz = _quat_rotate(jqx, jqy, jqz, mqw, ptx, pty, ptz)
            tx, ty, tz = ntx + rpx, nty + rpy, ntz + rpz
            qx, qy, qz, qw = _quat_mul(jqx, jqy, jqz, mqw, pqx, pqy, pqz, pqw)

            sign = jnp.where(qw < 0.0, -1.0, 1.0)
            qx, qy, qz, qw = qx * sign, qy * sign, qz * sign, qw * sign
            n2 = qx * qx + qy * qy + qz * qz
            n = (n2 + 1e-24) * _rsqrt(n2 + 1e-24)
            mn = jnp.minimum(n, qw)
            mx_ = jnp.maximum(n, qw)
            t_at = _atan01(mn / mx_)
            half = jnp.where(n > qw, jnp.float32(jnp.pi / 2) - t_at, t_at)
            theta = 2.0 * half
            small = n < 1e-6
            fnum = jnp.where(small, 2.0, theta)
            fden = jnp.where(small, jnp.maximum(qw, 1e-6), n)
            factor = fnum / fden
            phx, phy, phz = factor * qx, factor * qy, factor * qz

            th2 = phx * phx + phy * phy + phz * phz
            th2_safe = jnp.where(th2 < 1e-12, 1.0, th2)
            cot_term = half * qw / jnp.maximum(n, 1e-8)
            coef = jnp.where(theta < 1e-4, jnp.float32(1.0 / 12.0),
                             (1.0 - cot_term) / th2_safe)
            p1x, p1y, p1z = _cross(phx, phy, phz, tx, ty, tz)
            p2x, p2y, p2z = _cross(phx, phy, phz, p1x, p1y, p1z)
            taux = tx - 0.5 * p1x + coef * p2x
            tauy = ty - 0.5 * p1y + coef * p2y
            tauz = tz - 0.5 * p1z + coef * p2z

            obase = g * (LANES * 6) + lane * 6
            vals = (taux * cols(w_v, 0), tauy * cols(w_v, 1), tauz * cols(w_v, 2),
                    phx * cols(w_v, 3), phy * cols(w_v, 4), phz * cols(w_v, 5))
            for k, v in enumerate(vals):
                plsc.store_scatter(out_v, [obase + k], v)
            return carry2

        lax.fori_loop(0, GROUPS, group_body, 0, unroll=2)

    issue_idx(0, 0)
    issue_idx(1, 1)
    issue_mw(0, 0)
    issue_mw(1, 1)
    wait_idx(0)
    issue_gathers(0, 0)

    wait_idx(1)
    issue_gathers(1, 1)
    wait_gathers(0)
    wait_mw(0)
    issue_idx(2, 0)
    compute(0)
    issue_store(0, 0)
    issue_mw(2, 0)

    wait_idx(0)
    issue_gathers(2, 0)
    wait_gathers(1)
    wait_mw(1)
    issue_idx(3, 1)
    compute(1)
    issue_store(1, 1)
    issue_mw(3, 1)

    def pair_body(k, carry):
        for b in (0, 1):
            i = 2 * k + 2 + b
            p = b
            wait_idx(1 - p)
            issue_gathers(i + 1, 1 - p)
            wait_gathers(p)
            wait_mw(p)
            wait_store(p)
            issue_idx(i + 2, p)
            compute(p)
            issue_store(i, p)
            issue_mw(i + 2, p)
        return carry

    lax.fori_loop(0, (T_ITER - 2) // 2, pair_body, 0, unroll=False)

    wait_gathers(0)
    wait_mw(0)
    wait_idx(1)
    wait_mw(1)
    wait_store(0)
    wait_store(1)


W_CONV = 3200
N_CCHUNK = N_EDGES // W_CONV
T_CONV = (N_CCHUNK + N_WORKERS - 1) // N_WORKERS


def _conv_body(meas_t, wts_t, mout, wout, mv0, mv1, wv0, wv1, rd0, rd1, wr0, wr1):
    MV, WV, RD, WR = (mv0, mv1), (wv0, wv1), (rd0, rd1), (wr0, wr1)
    wid = lax.axis_index("s") * 2 + lax.axis_index("c")

    def cb(i):
        return jnp.minimum(wid + i * N_WORKERS, N_CCHUNK - 1) * W_CONV

    def issue_read(i, p):
        base = cb(i)
        pltpu.async_copy(meas_t.at[:, pl.ds(base, W_CONV)], MV[p].at[pl.ds(0, 7)], RD[p])
        pltpu.async_copy(wts_t.at[:, pl.ds(base, W_CONV)], WV[p].at[pl.ds(0, 6)], RD[p])

    def wait_read(p):
        pltpu.make_async_copy(meas_t.at[:, pl.ds(0, W_CONV)], MV[p].at[pl.ds(0, 7)], RD[p]).wait()
        pltpu.make_async_copy(wts_t.at[:, pl.ds(0, W_CONV)], WV[p].at[pl.ds(0, 6)], RD[p]).wait()

    def issue_write(i, p):
        base = cb(i)
        for k in range(7):
            pltpu.async_copy(MV[p].at[k], mout.at[pl.ds(k * N_EDGES + base, W_CONV)], WR[p])
        for k in range(6):
            pltpu.async_copy(WV[p].at[k], wout.at[pl.ds(k * N_EDGES + base, W_CONV)], WR[p])

    def wait_write(p):
        for k in range(7):
            pltpu.make_async_copy(meas_t.at[0, pl.ds(0, W_CONV)], MV[p].at[k], WR[p]).wait()
        for k in range(6):
            pltpu.make_async_copy(wts_t.at[0, pl.ds(0, W_CONV)], WV[p].at[k], WR[p]).wait()

    issue_read(0, 0)
    issue_read(1, 1)

    def conv_pair(k, carry):
        for b in (0, 1):
            i = 2 * k + b
            p = b
            wait_read(p)
            if True:
                pass
            issue_write(i, p)
            wait_write(p)
            issue_read(i + 2, p)
        return carry

    lax.fori_loop(0, T_CONV // 2, conv_pair, 0, unroll=False)
    wait_read(0)
    wait_read(1)


@jax.jit
def _pose_graph_conv(meas_t, wts_t):
    mesh = plsc.VectorSubcoreMesh(core_axis_name="c", subcore_axis_name="s")
    f = pl.kernel(
        _conv_body,
        out_type=(jax.ShapeDtypeStruct((7 * N_EDGES,), jnp.float32),
                  jax.ShapeDtypeStruct((6 * N_EDGES,), jnp.float32)),
        mesh=mesh,
        compiler_params=pltpu.CompilerParams(
            needs_layout_passes=False, use_tc_tiling_on_sc=True),
        scratch_types=[
            pltpu.VMEM((8, W_CONV), jnp.float32), pltpu.VMEM((8, W_CONV), jnp.float32),
            pltpu.VMEM((8, W_CONV), jnp.float32), pltpu.VMEM((8, W_CONV), jnp.float32),
            pltpu.SemaphoreType.DMA, pltpu.SemaphoreType.DMA,
            pltpu.SemaphoreType.DMA, pltpu.SemaphoreType.DMA,
        ],
    )
    return f(meas_t, wts_t)


@jax.jit
def _pose_graph_sc(table, src_idx, dst_idx, meas, wts):
    mesh = plsc.VectorSubcoreMesh(core_axis_name="c", subcore_axis_name="s")
    f = pl.kernel(
        _sc_body,
        out_type=jax.ShapeDtypeStruct((N_EDGES * 6,), jnp.float32),
        mesh=mesh,
        compiler_params=pltpu.CompilerParams(
            needs_layout_passes=False, use_tc_tiling_on_sc=False),
        scratch_types=[
            pltpu.VMEM((CHUNK,), jnp.int32), pltpu.VMEM((CHUNK,), jnp.int32),
            pltpu.VMEM((CHUNK,), jnp.int32), pltpu.VMEM((CHUNK,), jnp.int32),
            pltpu.VMEM((CHUNK, 8), jnp.float32), pltpu.VMEM((CHUNK, 8), jnp.float32),
            pltpu.VMEM((CHUNK, 8), jnp.float32), pltpu.VMEM((CHUNK, 8), jnp.float32),
            pltpu.VMEM((7, CHUNK), jnp.float32), pltpu.VMEM((7, CHUNK), jnp.float32),
            pltpu.VMEM((6, CHUNK), jnp.float32), pltpu.VMEM((6, CHUNK), jnp.float32),
            pltpu.VMEM((CHUNK * 6,), jnp.float32), pltpu.VMEM((CHUNK * 6,), jnp.float32),
            pltpu.SemaphoreType.DMA, pltpu.SemaphoreType.DMA,
            pltpu.SemaphoreType.DMA, pltpu.SemaphoreType.DMA,
            pltpu.SemaphoreType.DMA, pltpu.SemaphoreType.DMA,
            pltpu.SemaphoreType.DMA, pltpu.SemaphoreType.DMA,
        ],
    )
    return f(table, src_idx, dst_idx, meas, wts)


def kernel(poses_rest, edge_meas, edge_weights, edges_src, edges_dst):
    pose_0 = jnp.zeros((1, 7), jnp.float32).at[0, 6].set(1.0)
    all_poses = jnp.concatenate([pose_0, poses_rest], axis=0)
    table = jnp.pad(all_poses, ((0, 0), (0, 1)))
    meas_soa, wts_soa = _pose_graph_conv(edge_meas.T, edge_weights.T)
    return _pose_graph_sc(
        table,
        edges_src.astype(jnp.int32),
        edges_dst.astype(jnp.int32),
        meas_soa,
        wts_soa,
    )


# speedup vs baseline: 19.3072x; 1.0527x over previous
import jax
import jax.numpy as jnp
from jax import lax
from jax.experimental import pallas as pl
from jax.experimental.pallas import tpu as pltpu
from jax.experimental.pallas import tpu_sc as plsc

N_POSES = 100000
N_EDGES = 1600000
CHUNK = 512
N_CHUNKS = N_EDGES // CHUNK
N_WORKERS = 32
T_ITER = (N_CHUNKS + N_WORKERS - 1) // N_WORKERS
GATHER_SUB = 128
LANES = 16
GROUPS = CHUNK // LANES

_ATAN_C = (
    0.9999994931983208,
    -0.333277214292507,
    0.19897338970248968,
    -0.1356213417992754,
    0.08545195387626983,
    -0.03853451958906273,
    0.008408163058466886,
)


def _rsqrt(x):
    i = plsc.bitcast(x, jnp.int32)
    i = jnp.int32(0x5F3759DF) - lax.shift_right_logical(i, 1)
    y = plsc.bitcast(i, jnp.float32)
    half = 0.5 * x
    for _ in range(2):
        y = y * (1.5 - half * y * y)
    return y


def _atan01(a):
    u = a * a
    p = jnp.full_like(a, _ATAN_C[-1])
    for c in _ATAN_C[-2::-1]:
        p = p * u + c
    return a * p


def _cross(ax, ay, az, bx, by, bz):
    return (ay * bz - az * by, az * bx - ax * bz, ax * by - ay * bx)


def _quat_rotate(qx, qy, qz, qw, vx, vy, vz):
    tx, ty, tz = _cross(qx, qy, qz, vx, vy, vz)
    tx, ty, tz = 2.0 * tx, 2.0 * ty, 2.0 * tz
    cx, cy, cz = _cross(qx, qy, qz, tx, ty, tz)
    return (vx + qw * tx + cx, vy + qw * ty + cy, vz + qw * tz + cz)


def _quat_mul(x1, y1, z1, w1, x2, y2, z2, w2):
    x = w1 * x2 + x1 * w2 + y1 * z2 - z1 * y2
    y = w1 * y2 - x1 * z2 + y1 * w2 + z1 * x2
    z = w1 * z2 + x1 * y2 - y1 * x2 + z1 * w2
    w = w1 * w2 - x1 * x2 - y1 * y2 - z1 * z2
    return x, y, z, w


def _sc_body(table, src_idx, dst_idx, meas, wts, out,
             si0, si1, di0, di1, sv0, sv1, dv0, dv1,
             mv0, mv1, wv0, wv1, ov0, ov1,
             ix0, ix1, gt0, gt1, mw0, mw1, st0, st1):
    SI, DI = (si0, si1), (di0, di1)
    SV, DV = (sv0, sv1), (dv0, dv1)
    MV, WV = (mv0, mv1), (wv0, wv1)
    OV = (ov0, ov1)
    IX, GT, MW, ST = (ix0, ix1), (gt0, gt1), (mw0, mw1), (st0, st1)

    wid = lax.axis_index("s") * 2 + lax.axis_index("c")
    lane = lax.iota(jnp.int32, LANES)

    def cbase(i):
        c = jnp.minimum(wid + i * N_WORKERS, N_CHUNKS - 1)
        return c * CHUNK

    def issue_idx(i, p):
        base = cbase(i)
        pltpu.async_copy(src_idx.at[pl.ds(base, CHUNK)], SI[p], IX[p])
        pltpu.async_copy(dst_idx.at[pl.ds(base, CHUNK)], DI[p], IX[p])

    def wait_idx(p):
        pltpu.make_async_copy(src_idx.at[pl.ds(0, CHUNK)], SI[p], IX[p]).wait()
        pltpu.make_async_copy(dst_idx.at[pl.ds(0, CHUNK)], DI[p], IX[p]).wait()

    def issue_gathers(i, p):
        for j in range(CHUNK // GATHER_SUB):
            sl = pl.ds(j * GATHER_SUB, GATHER_SUB)
            pltpu.async_copy(table.at[SI[p].at[sl]], SV[p].at[sl], GT[p])
            pltpu.async_copy(table.at[DI[p].at[sl]], DV[p].at[sl], GT[p])

    def wait_gathers(p):
        for j in range(CHUNK // GATHER_SUB):
            sl = pl.ds(j * GATHER_SUB, GATHER_SUB)
            pltpu.make_async_copy(table.at[pl.ds(0, GATHER_SUB)], SV[p].at[sl], GT[p]).wait()
            pltpu.make_async_copy(table.at[pl.ds(0, GATHER_SUB)], DV[p].at[sl], GT[p]).wait()

    def issue_mw(i, p):
        base = cbase(i)
        for k in range(7):
            pltpu.async_copy(meas.at[pl.ds(k * N_EDGES + base, CHUNK)],
                             MV[p].at[k], MW[p])
        for k in range(6):
            pltpu.async_copy(wts.at[pl.ds(k * N_EDGES + base, CHUNK)],
                             WV[p].at[k], MW[p])

    def wait_mw(p):
        for k in range(7):
            pltpu.make_async_copy(meas.at[pl.ds(0, CHUNK)], MV[p].at[k], MW[p]).wait()
        for k in range(6):
            pltpu.make_async_copy(wts.at[pl.ds(0, CHUNK)], WV[p].at[k], MW[p]).wait()

    def issue_store(i, p):
        pltpu.async_copy(OV[p], out.at[pl.ds(cbase(i) * 6, CHUNK * 6)], ST[p])

    def wait_store(p):
        pltpu.make_async_copy(out.at[pl.ds(0, CHUNK * 6)], OV[p], ST[p]).wait()

    def compute(p):
        src_v, dst_v, meas_v, w_v, out_v = SV[p], DV[p], MV[p], WV[p], OV[p]

        def group_body(g, carry2):
            rows = g * LANES + lane

            def col(ref, k):
                return plsc.load_gather(ref, [rows, jnp.full((LANES,), k, jnp.int32)])

            def cols(ref, k):
                return ref[k, pl.ds(g * LANES, LANES)]

            stx, sty, stz = col(src_v, 0), col(src_v, 1), col(src_v, 2)
            sqx, sqy, sqz, sqw = col(src_v, 3), col(src_v, 4), col(src_v, 5), col(src_v, 6)
            dtx, dty, dtz = col(dst_v, 0), col(dst_v, 1), col(dst_v, 2)
            dqx, dqy, dqz, dqw = col(dst_v, 3), col(dst_v, 4), col(dst_v, 5), col(dst_v, 6)
            mtx, mty, mtz = cols(meas_v, 0), cols(meas_v, 1), cols(meas_v, 2)
            mqx, mqy, mqz, mqw = (cols(meas_v, 3), cols(meas_v, 4),
                                  cols(meas_v, 5), cols(meas_v, 6))

            iqx, iqy, iqz = -sqx, -sqy, -sqz
            rx, ry, rz = _quat_rotate(iqx, iqy, iqz, sqw, stx, sty, stz)
            itx, ity, itz = -rx, -ry, -rz
            rdx, rdy, rdz = _quat_rotate(iqx, iqy, iqz, sqw, dtx, dty, dtz)
            ptx, pty, ptz = itx + rdx, ity + rdy, itz + rdz
            pqx, pqy, pqz, pqw = _quat_mul(iqx, iqy, iqz, sqw, dqx, dqy, dqz, dqw)
            jqx, jqy, jqz = -mqx, -mqy, -mqz
            mx, my, mz = _quat_rotate(jqx, jqy, jqz, mqw, mtx, mty, mtz)
            ntx, nty, ntz = -mx, -my, -mz
            rpx, rpy, rpz = _quat_rotate(jqx, jqy, jqz, mqw, ptx, pty, ptz)
            tx, ty, tz = ntx + rpx, nty + rpy, ntz + rpz
            qx, qy, qz, qw = _quat_mul(jqx, jqy, jqz, mqw, pqx, pqy, pqz, pqw)

            sign = jnp.where(qw < 0.0, -1.0, 1.0)
            qx, qy, qz, qw = qx * sign, qy * sign, qz * sign, qw * sign
            n2 = qx * qx + qy * qy + qz * qz
            n = (n2 + 1e-24) * _rsqrt(n2 + 1e-24)
            mn = jnp.minimum(n, qw)
            mx_ = jnp.maximum(n, qw)
            t_at = _atan01(mn / mx_)
            half = jnp.where(n > qw, jnp.float32(jnp.pi / 2) - t_at, t_at)
            theta = 2.0 * half
            small = n < 1e-6
            fnum = jnp.where(small, 2.0, theta)
            fden = jnp.where(small, jnp.maximum(qw, 1e-6), n)
            inv_den = 1.0 / fden
            factor = fnum * inv_den
            phx, phy, phz = factor * qx, factor * qy, factor * qz

            th2 = phx * phx + phy * phy + phz * phz
            th2_safe = jnp.where(th2 < 1e-12, 1.0, th2)
            cot_term = half * qw * inv_den
            coef = jnp.where(theta < 1e-4, jnp.float32(1.0 / 12.0),
                             (1.0 - cot_term) / th2_safe)
            p1x, p1y, p1z = _cross(phx, phy, phz, tx, ty, tz)
            p2x, p2y, p2z = _cross(phx, phy, phz, p1x, p1y, p1z)
            taux = tx - 0.5 * p1x + coef * p2x
            tauy = ty - 0.5 * p1y + coef * p2y
            tauz = tz - 0.5 * p1z + coef * p2z

            obase = g * (LANES * 6) + lane * 6
            vals = (taux * cols(w_v, 0), tauy * cols(w_v, 1), tauz * cols(w_v, 2),
                    phx * cols(w_v, 3), phy * cols(w_v, 4), phz * cols(w_v, 5))
            for k, v in enumerate(vals):
                plsc.store_scatter(out_v, [obase + k], v)
            return carry2

        lax.fori_loop(0, GROUPS, group_body, 0, unroll=4)

    issue_idx(0, 0)
    issue_idx(1, 1)
    issue_mw(0, 0)
    issue_mw(1, 1)
    wait_idx(0)
    issue_gathers(0, 0)

    wait_idx(1)
    issue_gathers(1, 1)
    wait_gathers(0)
    wait_mw(0)
    issue_idx(2, 0)
    compute(0)
    issue_store(0, 0)
    issue_mw(2, 0)

    wait_idx(0)
    issue_gathers(2, 0)
    wait_gathers(1)
    wait_mw(1)
    issue_idx(3, 1)
    compute(1)
    issue_store(1, 1)
    issue_mw(3, 1)

    def pair_body(k, carry):
        for b in (0, 1):
            i = 2 * k + 2 + b
            p = b
            wait_idx(1 - p)
            issue_gathers(i + 1, 1 - p)
            wait_gathers(p)
            wait_mw(p)
            wait_store(p)
            issue_idx(i + 2, p)
            compute(p)
            issue_store(i, p)
            issue_mw(i + 2, p)
        return carry

    lax.fori_loop(0, (T_ITER - 2) // 2, pair_body, 0, unroll=False)

    wait_gathers(0)
    wait_mw(0)
    wait_idx(1)
    wait_mw(1)
    wait_store(0)
    wait_store(1)


W_CONV = 3200
N_CCHUNK = N_EDGES // W_CONV
T_CONV = (N_CCHUNK + N_WORKERS - 1) // N_WORKERS


def _conv_body(meas_t, wts_t, mout, wout, mv0, mv1, wv0, wv1, rd0, rd1, wr0, wr1):
    MV, WV, RD, WR = (mv0, mv1), (wv0, wv1), (rd0, rd1), (wr0, wr1)
    wid = lax.axis_index("s") * 2 + lax.axis_index("c")

    def cb(i):
        return jnp.minimum(wid + i * N_WORKERS, N_CCHUNK - 1) * W_CONV

    def issue_read(i, p):
        base = cb(i)
        pltpu.async_copy(meas_t.at[:, pl.ds(base, W_CONV)], MV[p].at[pl.ds(0, 7)], RD[p])
        pltpu.async_copy(wts_t.at[:, pl.ds(base, W_CONV)], WV[p].at[pl.ds(0, 6)], RD[p])

    def wait_read(p):
        pltpu.make_async_copy(meas_t.at[:, pl.ds(0, W_CONV)], MV[p].at[pl.ds(0, 7)], RD[p]).wait()
        pltpu.make_async_copy(wts_t.at[:, pl.ds(0, W_CONV)], WV[p].at[pl.ds(0, 6)], RD[p]).wait()

    def issue_write(i, p):
        base = cb(i)
        for k in range(7):
            pltpu.async_copy(MV[p].at[k], mout.at[pl.ds(k * N_EDGES + base, W_CONV)], WR[p])
        for k in range(6):
            pltpu.async_copy(WV[p].at[k], wout.at[pl.ds(k * N_EDGES + base, W_CONV)], WR[p])

    def wait_write(p):
        for k in range(7):
            pltpu.make_async_copy(meas_t.at[0, pl.ds(0, W_CONV)], MV[p].at[k], WR[p]).wait()
        for k in range(6):
            pltpu.make_async_copy(wts_t.at[0, pl.ds(0, W_CONV)], WV[p].at[k], WR[p]).wait()

    issue_read(0, 0)
    issue_read(1, 1)

    def conv_pair(k, carry):
        for b in (0, 1):
            i = 2 * k + b
            p = b
            wait_read(p)
            if True:
                pass
            issue_write(i, p)
            wait_write(p)
            issue_read(i + 2, p)
        return carry

    lax.fori_loop(0, T_CONV // 2, conv_pair, 0, unroll=False)
    wait_read(0)
    wait_read(1)


@jax.jit
def _pose_graph_conv(meas_t, wts_t):
    mesh = plsc.VectorSubcoreMesh(core_axis_name="c", subcore_axis_name="s")
    f = pl.kernel(
        _conv_body,
        out_type=(jax.ShapeDtypeStruct((7 * N_EDGES,), jnp.float32),
                  jax.ShapeDtypeStruct((6 * N_EDGES,), jnp.float32)),
        mesh=mesh,
        compiler_params=pltpu.CompilerParams(
            needs_layout_passes=False, use_tc_tiling_on_sc=True),
        scratch_types=[
            pltpu.VMEM((8, W_CONV), jnp.float32), pltpu.VMEM((8, W_CONV), jnp.float32),
            pltpu.VMEM((8, W_CONV), jnp.float32), pltpu.VMEM((8, W_CONV), jnp.float32),
            pltpu.SemaphoreType.DMA, pltpu.SemaphoreType.DMA,
            pltpu.SemaphoreType.DMA, pltpu.SemaphoreType.DMA,
        ],
    )
    return f(meas_t, wts_t)


@jax.jit
def _pose_graph_sc(table, src_idx, dst_idx, meas, wts):
    mesh = plsc.VectorSubcoreMesh(core_axis_name="c", subcore_axis_name="s")
    f = pl.kernel(
        _sc_body,
        out_type=jax.ShapeDtypeStruct((N_EDGES * 6,), jnp.float32),
        mesh=mesh,
        compiler_params=pltpu.CompilerParams(
            needs_layout_passes=False, use_tc_tiling_on_sc=False),
        scratch_types=[
            pltpu.VMEM((CHUNK,), jnp.int32), pltpu.VMEM((CHUNK,), jnp.int32),
            pltpu.VMEM((CHUNK,), jnp.int32), pltpu.VMEM((CHUNK,), jnp.int32),
            pltpu.VMEM((CHUNK, 8), jnp.float32), pltpu.VMEM((CHUNK, 8), jnp.float32),
            pltpu.VMEM((CHUNK, 8), jnp.float32), pltpu.VMEM((CHUNK, 8), jnp.float32),
            pltpu.VMEM((7, CHUNK), jnp.float32), pltpu.VMEM((7, CHUNK), jnp.float32),
            pltpu.VMEM((6, CHUNK), jnp.float32), pltpu.VMEM((6, CHUNK), jnp.float32),
            pltpu.VMEM((CHUNK * 6,), jnp.float32), pltpu.VMEM((CHUNK * 6,), jnp.float32),
            pltpu.SemaphoreType.DMA, pltpu.SemaphoreType.DMA,
            pltpu.SemaphoreType.DMA, pltpu.SemaphoreType.DMA,
            pltpu.SemaphoreType.DMA, pltpu.SemaphoreType.DMA,
            pltpu.SemaphoreType.DMA, pltpu.SemaphoreType.DMA,
        ],
    )
    return f(table, src_idx, dst_idx, meas, wts)


def kernel(poses_rest, edge_meas, edge_weights, edges_src, edges_dst):
    pose_0 = jnp.zeros((1, 7), jnp.float32).at[0, 6].set(1.0)
    all_poses = jnp.concatenate([pose_0, poses_rest], axis=0)
    table = jnp.pad(all_poses, ((0, 0), (0, 1)))
    meas_soa, wts_soa = _pose_graph_conv(edge_meas.T, edge_weights.T)
    return _pose_graph_sc(
        table,
        edges_src.astype(jnp.int32),
        edges_dst.astype(jnp.int32),
        meas_soa,
        wts_soa,
    )
